# baseline (speedup 1.0000x reference)
import jax
import jax.numpy as jnp
from jax.experimental import pallas as pl
from jax.experimental.pallas import tpu as pltpu

B = 4096
D = 3072
H1 = 1024
ENC_OUT = 512
P1 = 256
P2 = 128
EMB = 2
EMB_PAD = 128

ROWS = 1024
KBLK = 1024
NK = D // KBLK


def _fwd_kernel(x_ref, xs_ref, w1_ref, b1_ref, w2_ref, b2_ref,
                wp1_ref, bp1_ref, wp2_ref, bp2_ref, wp3_ref, bp3_ref,
                reps_ref, emb_ref, sreps_ref, semb_ref,
                accx_ref, accs_ref):
    k = pl.program_id(1)
    w1 = w1_ref[...]
    px = jnp.dot(x_ref[...], w1, preferred_element_type=jnp.float32)
    ps = jnp.dot(xs_ref[...], w1, preferred_element_type=jnp.float32)

    @pl.when(k == 0)
    def _():
        accx_ref[...] = px
        accs_ref[...] = ps

    @pl.when(jnp.logical_and(k > 0, k < NK - 1))
    def _():
        accx_ref[...] += px
        accs_ref[...] += ps

    @pl.when(k == NK - 1)
    def _():
        def tail(h_pre, reps_out, emb_out):
            h = jnp.maximum(h_pre + b1_ref[...], 0.0)
            reps = jnp.maximum(
                jnp.dot(h, w2_ref[...], preferred_element_type=jnp.float32)
                + b2_ref[...], 0.0)
            reps_out[...] = reps
            e = jnp.maximum(
                jnp.dot(reps, wp1_ref[...], preferred_element_type=jnp.float32)
                + bp1_ref[...], 0.0)
            e = jnp.maximum(
                jnp.dot(e, wp2_ref[...], preferred_element_type=jnp.float32)
                + bp2_ref[...], 0.0)
            emb_out[...] = jnp.dot(
                e, wp3_ref[...], preferred_element_type=jnp.float32) + bp3_ref[...]

        tail(accx_ref[...] + px, reps_ref, emb_ref)
        tail(accs_ref[...] + ps, sreps_ref, semb_ref)


def kernel(x, x_sim, W1, b1, W2, b2, Wp1, bp1, Wp2, bp2, Wp3, bp3):
    Wp3p = jnp.zeros((P2, EMB_PAD), dtype=Wp3.dtype).at[:, :EMB].set(Wp3)
    bp3p = jnp.zeros((EMB_PAD,), dtype=bp3.dtype).at[:EMB].set(bp3)

    grid = (B // ROWS, NK)
    in_row_spec = pl.BlockSpec((ROWS, KBLK), lambda i, k: (i, k))
    w1_spec = pl.BlockSpec((KBLK, H1), lambda i, k: (k, 0))
    out_row_spec = lambda w: pl.BlockSpec((ROWS, w), lambda i, k: (i, 0))
    full_spec = lambda r, c: pl.BlockSpec((r, c), lambda i, k: (0, 0))
    bias_spec = lambda w: pl.BlockSpec((1, w), lambda i, k: (0, 0))

    out_shapes = (
        jax.ShapeDtypeStruct((B, ENC_OUT), jnp.float32),
        jax.ShapeDtypeStruct((B, EMB_PAD), jnp.float32),
        jax.ShapeDtypeStruct((B, ENC_OUT), jnp.float32),
        jax.ShapeDtypeStruct((B, EMB_PAD), jnp.float32),
    )

    reps, emb_p, sreps, semb_p = pl.pallas_call(
        _fwd_kernel,
        grid=grid,
        in_specs=[
            in_row_spec, in_row_spec,
            w1_spec, bias_spec(H1),
            full_spec(H1, ENC_OUT), bias_spec(ENC_OUT),
            full_spec(ENC_OUT, P1), bias_spec(P1),
            full_spec(P1, P2), bias_spec(P2),
            full_spec(P2, EMB_PAD), bias_spec(EMB_PAD),
        ],
        out_specs=(
            out_row_spec(ENC_OUT), out_row_spec(EMB_PAD),
            out_row_spec(ENC_OUT), out_row_spec(EMB_PAD),
        ),
        out_shape=out_shapes,
        scratch_shapes=[
            pltpu.VMEM((ROWS, H1), jnp.float32),
            pltpu.VMEM((ROWS, H1), jnp.float32),
        ],
        compiler_params=pltpu.CompilerParams(
            dimension_semantics=("arbitrary", "arbitrary"),
        ),
    )(x, x_sim, W1, b1.reshape(1, H1), W2, b2.reshape(1, ENC_OUT),
      Wp1, bp1.reshape(1, P1), Wp2, bp2.reshape(1, P2),
      Wp3p, bp3p.reshape(1, EMB_PAD))

    return (reps, emb_p[:, :EMB], sreps, semb_p[:, :EMB])


# speedup vs baseline: 1.0880x; 1.0880x over previous
import jax
import jax.numpy as jnp
from jax.experimental import pallas as pl
from jax.experimental.pallas import tpu as pltpu

B = 4096
D = 3072
H1 = 1024
ENC_OUT = 512
P1 = 256
P2 = 128
EMB = 2
EMB_PAD = 128

ROWS = 512


def _fwd_kernel(x_ref, xs_ref, w1_ref, b1_ref, w2_ref, b2_ref,
                wp1_ref, bp1_ref, wp2_ref, bp2_ref, wp3_ref, bp3_ref,
                reps_ref, emb_ref, sreps_ref, semb_ref):
    w1 = w1_ref[...]
    w2 = w2_ref[...]
    wp1 = wp1_ref[...]
    wp2 = wp2_ref[...]
    wp3 = wp3_ref[...]

    def encode(inp, reps_out, emb_out):
        h = jnp.maximum(
            jnp.dot(inp, w1, preferred_element_type=jnp.float32) + b1_ref[...], 0.0)
        reps = jnp.maximum(
            jnp.dot(h, w2, preferred_element_type=jnp.float32) + b2_ref[...], 0.0)
        reps_out[...] = reps
        e = jnp.maximum(
            jnp.dot(reps, wp1, preferred_element_type=jnp.float32) + bp1_ref[...], 0.0)
        e = jnp.maximum(
            jnp.dot(e, wp2, preferred_element_type=jnp.float32) + bp2_ref[...], 0.0)
        emb_out[...] = jnp.dot(e, wp3, preferred_element_type=jnp.float32) + bp3_ref[...]

    encode(x_ref[...], reps_ref, emb_ref)
    encode(xs_ref[...], sreps_ref, semb_ref)


def kernel(x, x_sim, W1, b1, W2, b2, Wp1, bp1, Wp2, bp2, Wp3, bp3):
    Wp3p = jnp.zeros((P2, EMB_PAD), dtype=Wp3.dtype).at[:, :EMB].set(Wp3)
    bp3p = jnp.zeros((EMB_PAD,), dtype=bp3.dtype).at[:EMB].set(bp3)

    grid = (B // ROWS,)
    row_spec = lambda w: pl.BlockSpec((ROWS, w), lambda i: (i, 0))
    full_spec = lambda r, c: pl.BlockSpec((r, c), lambda i: (0, 0))
    bias_spec = lambda w: pl.BlockSpec((1, w), lambda i: (0, 0))

    out_shapes = (
        jax.ShapeDtypeStruct((B, ENC_OUT), jnp.float32),
        jax.ShapeDtypeStruct((B, EMB_PAD), jnp.float32),
        jax.ShapeDtypeStruct((B, ENC_OUT), jnp.float32),
        jax.ShapeDtypeStruct((B, EMB_PAD), jnp.float32),
    )

    reps, emb_p, sreps, semb_p = pl.pallas_call(
        _fwd_kernel,
        grid=grid,
        in_specs=[
            row_spec(D), row_spec(D),
            full_spec(D, H1), bias_spec(H1),
            full_spec(H1, ENC_OUT), bias_spec(ENC_OUT),
            full_spec(ENC_OUT, P1), bias_spec(P1),
            full_spec(P1, P2), bias_spec(P2),
            full_spec(P2, EMB_PAD), bias_spec(EMB_PAD),
        ],
        out_specs=(
            row_spec(ENC_OUT), row_spec(EMB_PAD),
            row_spec(ENC_OUT), row_spec(EMB_PAD),
        ),
        out_shape=out_shapes,
        compiler_params=pltpu.CompilerParams(
            dimension_semantics=("arbitrary",),
        ),
    )(x, x_sim, W1, b1.reshape(1, H1), W2, b2.reshape(1, ENC_OUT),
      Wp1, bp1.reshape(1, P1), Wp2, bp2.reshape(1, P2),
      Wp3p, bp3p.reshape(1, EMB_PAD))

    return (reps, emb_p[:, :EMB], sreps, semb_p[:, :EMB])


# speedup vs baseline: 1.1197x; 1.0291x over previous
import jax
import jax.numpy as jnp
from jax.experimental import pallas as pl
from jax.experimental.pallas import tpu as pltpu

B = 4096
D = 3072
H1 = 1024
ENC_OUT = 512
P1 = 256
P2 = 128
EMB = 2
EMB_PAD = 128

ROWS = 512


def _fwd_kernel(x_ref, xs_ref, w1_ref, b1_ref, w2_ref, b2_ref,
                wp1_ref, bp1_ref, wp2_ref, bp2_ref, wp3_ref, bp3_ref,
                reps_ref, emb_ref, sreps_ref, semb_ref):
    w1 = w1_ref[...]
    w2 = w2_ref[...]
    wp1 = wp1_ref[...]
    wp2 = wp2_ref[...]
    wp3 = wp3_ref[...]

    def encode(inp, reps_out, emb_out):
        h = jnp.maximum(
            jnp.dot(inp, w1, preferred_element_type=jnp.float32) + b1_ref[...], 0.0)
        reps = jnp.maximum(
            jnp.dot(h, w2, preferred_element_type=jnp.float32) + b2_ref[...], 0.0)
        reps_out[...] = reps
        e = jnp.maximum(
            jnp.dot(reps, wp1, preferred_element_type=jnp.float32) + bp1_ref[...], 0.0)
        e = jnp.maximum(
            jnp.dot(e, wp2, preferred_element_type=jnp.float32) + bp2_ref[...], 0.0)
        emb_out[...] = (jnp.dot(e, wp3, preferred_element_type=jnp.float32)
                        + bp3_ref[...])

    encode(x_ref[...], reps_ref, emb_ref)
    encode(xs_ref[...], sreps_ref, semb_ref)


def kernel(x, x_sim, W1, b1, W2, b2, Wp1, bp1, Wp2, bp2, Wp3, bp3):
    grid = (B // ROWS,)
    row_spec = lambda w: pl.BlockSpec((ROWS, w), lambda i: (i, 0))
    full_spec = lambda r, c: pl.BlockSpec((r, c), lambda i: (0, 0))
    bias_spec = lambda w: pl.BlockSpec((1, w), lambda i: (0, 0))

    out_shapes = (
        jax.ShapeDtypeStruct((B, ENC_OUT), jnp.float32),
        jax.ShapeDtypeStruct((B, EMB), jnp.float32),
        jax.ShapeDtypeStruct((B, ENC_OUT), jnp.float32),
        jax.ShapeDtypeStruct((B, EMB), jnp.float32),
    )

    reps, emb, sreps, semb = pl.pallas_call(
        _fwd_kernel,
        grid=grid,
        in_specs=[
            row_spec(D), row_spec(D),
            full_spec(D, H1), bias_spec(H1),
            full_spec(H1, ENC_OUT), bias_spec(ENC_OUT),
            full_spec(ENC_OUT, P1), bias_spec(P1),
            full_spec(P1, P2), bias_spec(P2),
            full_spec(P2, EMB), bias_spec(EMB),
        ],
        out_specs=(
            row_spec(ENC_OUT), row_spec(EMB),
            row_spec(ENC_OUT), row_spec(EMB),
        ),
        out_shape=out_shapes,
        compiler_params=pltpu.CompilerParams(
            dimension_semantics=("arbitrary",),
        ),
    )(x, x_sim, W1, b1.reshape(1, H1), W2, b2.reshape(1, ENC_OUT),
      Wp1, bp1.reshape(1, P1), Wp2, bp2.reshape(1, P2),
      Wp3, bp3.reshape(1, EMB))

    return (reps, emb, sreps, semb)
